# baseline (speedup 1.0000x reference)
import jax
import jax.numpy as jnp
from jax.experimental import pallas as pl
from jax.experimental.pallas import tpu as pltpu

HIDDEN = 2048
EXPERTS = 16
BT = 256
NBUF = 12
LOOKAHEAD = 10


def _router_block(x_hbm, wt_ref, logits_ref, probs_ref, pack_ref, xbuf, sems):
    i = pl.program_id(0)
    nblk = pl.num_programs(0)

    def issue(blk):
        slot = jax.lax.rem(blk, NBUF)
        pltpu.make_async_copy(
            x_hbm.at[pl.ds(blk * BT, BT), :],
            xbuf.at[slot],
            sems.at[slot],
        ).start()

    @pl.when(i == 0)
    def _():
        for k in range(LOOKAHEAD):
            issue(k)

    @pl.when(i + LOOKAHEAD < nblk)
    def _():
        issue(i + LOOKAHEAD)

    slot = jax.lax.rem(i, NBUF)
    pltpu.make_async_copy(
        x_hbm.at[pl.ds(i * BT, BT), :],
        xbuf.at[slot],
        sems.at[slot],
    ).wait()

    xb = xbuf[slot]
    wt = wt_ref[...]
    logits = jnp.dot(xb, wt, preferred_element_type=jnp.float32)
    logits_ref[...] = logits

    lt = logits.T
    m = jnp.max(lt, axis=0, keepdims=True)
    e = jnp.exp(lt - m)
    s = jnp.sum(e, axis=0, keepdims=True)
    r = 1.0 / s
    pt = e * r
    probs_ref[...] = pt.T

    plsum = jnp.sum(pt * lt, axis=0, keepdims=True)
    ent = m + jnp.log(s) - plsum

    rows = jax.lax.broadcasted_iota(jnp.int32, (EXPERTS, BT), 0).astype(jnp.float32)
    w1 = jnp.max(pt, axis=0, keepdims=True)
    i1 = jnp.min(jnp.where(pt == w1, rows, float(EXPERTS)), axis=0, keepdims=True)
    masked = jnp.where(rows == i1, -jnp.inf, pt)
    w2 = jnp.max(masked, axis=0, keepdims=True)
    i2 = jnp.min(jnp.where(masked == w2, rows, float(EXPERTS)), axis=0, keepdims=True)

    rt = 1.0 / (w1 + w2 + 1e-9)
    zero = jnp.zeros((3, BT), jnp.float32)
    strip = jnp.concatenate([w1 * rt, w2 * rt, i1, i2, ent, zero], axis=0)
    pack_ref[...] = strip.T


def kernel(x, W):
    b, s, h = x.shape
    T = b * s
    x_flat = x.reshape(T, h)
    wt = W.T

    grid = (T // BT,)
    out_shapes = (
        jax.ShapeDtypeStruct((T, EXPERTS), jnp.float32),
        jax.ShapeDtypeStruct((T, EXPERTS), jnp.float32),
        jax.ShapeDtypeStruct((T, 8), jnp.float32),
    )
    tok_spec = lambda w: pl.BlockSpec((BT, w), lambda i: (i, 0))
    logits, probs, pack = pl.pallas_call(
        _router_block,
        grid=grid,
        in_specs=[
            pl.BlockSpec(memory_space=pltpu.MemorySpace.HBM),
            pl.BlockSpec((HIDDEN, EXPERTS), lambda i: (0, 0)),
        ],
        out_specs=(
            tok_spec(EXPERTS),
            tok_spec(EXPERTS),
            tok_spec(8),
        ),
        out_shape=out_shapes,
        scratch_shapes=[
            pltpu.MemorySpace.VMEM((NBUF, BT, HIDDEN), jnp.float32),
            pltpu.SemaphoreType.DMA((NBUF,)),
        ],
        compiler_params=pltpu.CompilerParams(
            dimension_semantics=("arbitrary",),
        ),
    )(x_flat, wt)

    tw = pack[:, 0:2]
    ti = pack[:, 2:4].astype(jnp.int32)
    entropy = pack[:, 4]
    return (tw, ti, probs, probs, logits, entropy)
